# Initial kernel scaffold; baseline (speedup 1.0000x reference)
import functools

import jax
import jax.numpy as jnp
from jax import lax
from jax.experimental import pallas as pl
from jax.experimental.pallas import tpu as pltpu
from jax.experimental.pallas import tpu_sc as plsc

N = 10000
E_MESH = 320000
E_WORLD = 160000
D = 128
H = 128

NC = 2
NS = 16
ACC_ROWS = 10240
ACC_W = 144
ROWS_PER_TILE = ACC_ROWS // NS
CHUNK = 128

MESH_PER_TILE = E_MESH // NS
WORLD_PER_TILE = E_WORLD // NS


def _sc_scatter(mesh_attr, mesh_dst, world_attr, world_dst, zeros_init, ones_col):
    mesh = plsc.VectorSubcoreMesh(
        core_axis_name="c", subcore_axis_name="s", num_cores=NC, num_subcores=NS
    )

    @functools.partial(
        pl.kernel,
        out_type=(
            jax.ShapeDtypeStruct((ACC_ROWS, ACC_W), jnp.float32),
            jax.ShapeDtypeStruct((ACC_ROWS, ACC_W), jnp.float32),
        ),
        mesh=mesh,
        scratch_types=[
            pltpu.VMEM_SHARED((ACC_ROWS, ACC_W), jnp.float32),
            pltpu.VMEM((CHUNK, ACC_W), jnp.float32),
            pltpu.VMEM((CHUNK,), jnp.int32),
            pltpu.VMEM((32,), jnp.int32),
            pltpu.VMEM((16,), jnp.int32),
        ],
    )
    def k(mesh_attr_h, mesh_dst_h, world_attr_h, world_dst_h, zeros_h, ones_h,
          mesh_out, world_out, acc_sh, buf, idx, idx_r32, idx_r16):
        c = lax.axis_index("c")
        s = lax.axis_index("s")
        row0 = s * ROWS_PER_TILE

        pltpu.sync_copy(zeros_h, acc_sh.at[pl.ds(row0, ROWS_PER_TILE)])
        pltpu.sync_copy(ones_h, buf)
        plsc.subcore_barrier()

        def edge_loop(attr_h, dst_h, per_tile, idx_rem):
            base = s * per_tile
            n_full = per_tile // CHUNK
            rem = per_tile % CHUNK

            def body(i, carry):
                off = base + i * CHUNK
                pltpu.sync_copy(attr_h.at[pl.ds(off, CHUNK)],
                                buf.at[:, pl.ds(0, D)])
                pltpu.sync_copy(dst_h.at[pl.ds(off, CHUNK)], idx)
                pltpu.sync_copy(buf, acc_sh.at[idx], add=True)
                return carry

            lax.fori_loop(0, n_full, body, 0)
            if rem:
                off = base + n_full * CHUNK
                pltpu.sync_copy(attr_h.at[pl.ds(off, rem)],
                                buf.at[pl.ds(0, rem), pl.ds(0, D)])
                pltpu.sync_copy(dst_h.at[pl.ds(off, rem)], idx_rem)
                pltpu.sync_copy(buf.at[pl.ds(0, rem)], acc_sh.at[idx_rem],
                                add=True)

        @pl.when(c == 0)
        def _():
            edge_loop(mesh_attr_h, mesh_dst_h, MESH_PER_TILE, idx_r32)

        @pl.when(c == 1)
        def _():
            edge_loop(world_attr_h, world_dst_h, WORLD_PER_TILE, idx_r16)

        plsc.subcore_barrier()

        @pl.when(c == 0)
        def _():
            pltpu.sync_copy(acc_sh.at[pl.ds(row0, ROWS_PER_TILE)],
                            mesh_out.at[pl.ds(row0, ROWS_PER_TILE)])

        @pl.when(c == 1)
        def _():
            pltpu.sync_copy(acc_sh.at[pl.ds(row0, ROWS_PER_TILE)],
                            world_out.at[pl.ds(row0, ROWS_PER_TILE)])

    return k(mesh_attr, mesh_dst, world_attr, world_dst, zeros_init, ones_col)


def _tc_body(x_ref, am_ref, aw_ref, w1_ref, b1_ref, w2_ref, b2_ref,
             g_ref, be_ref, o_ref):
    xv = x_ref[...]
    am = am_ref[:, :D] * (1.0 / jnp.maximum(am_ref[:, D:D + 1], 1.0))
    aw = aw_ref[:, :D] * (1.0 / jnp.maximum(aw_ref[:, D:D + 1], 1.0))
    w1 = w1_ref[...]
    hp = jax.lax.Precision.HIGHEST
    h1 = (jnp.dot(xv, w1[0:D], precision=hp)
          + jnp.dot(am, w1[D:2 * D], precision=hp)
          + jnp.dot(aw, w1[2 * D:3 * D], precision=hp)
          + b1_ref[...])
    h1 = jnp.maximum(h1, 0.0)
    h2 = jnp.dot(h1, w2_ref[...], precision=hp) + b2_ref[...]
    mu = jnp.mean(h2, axis=1, keepdims=True)
    dlt = h2 - mu
    var = jnp.mean(dlt * dlt, axis=1, keepdims=True)
    o_ref[...] = xv + dlt * jax.lax.rsqrt(var + 1e-5) * g_ref[...] + be_ref[...]


def _tc_mlp(x, mesh_acc, world_acc, W1, b1, W2, b2, g, b):
    BR = 512
    grid = (ACC_ROWS // BR,)
    full = lambda shape: pl.BlockSpec(shape, lambda i: (0, 0))
    return pl.pallas_call(
        _tc_body,
        grid=grid,
        in_specs=[
            pl.BlockSpec((BR, D), lambda i: (i, 0)),
            pl.BlockSpec((BR, ACC_W), lambda i: (i, 0)),
            pl.BlockSpec((BR, ACC_W), lambda i: (i, 0)),
            full((3 * D, H)),
            full((1, H)),
            full((H, D)),
            full((1, D)),
            full((1, D)),
            full((1, D)),
        ],
        out_specs=pl.BlockSpec((BR, D), lambda i: (i, 0)),
        out_shape=jax.ShapeDtypeStruct((N, D), jnp.float32),
    )(x, mesh_acc, world_acc, W1, b1, W2, b2, g, b)


def kernel(x, mesh_edge_attr, mesh_edge_index, world_edge_attr,
           world_edge_index, W1, b1, W2, b2, ln_gamma, ln_beta):
    mesh_dst = mesh_edge_index[1].astype(jnp.int32)
    world_dst = world_edge_index[1].astype(jnp.int32)
    zeros_init = jnp.zeros((ROWS_PER_TILE, ACC_W), jnp.float32)
    ones_col = jnp.zeros((CHUNK, ACC_W), jnp.float32).at[:, D].set(1.0)

    mesh_acc, world_acc = _sc_scatter(
        mesh_edge_attr, mesh_dst, world_edge_attr, world_dst,
        zeros_init, ones_col)

    return _tc_mlp(
        x, mesh_acc, world_acc, W1, b1.reshape(1, H), W2, b2.reshape(1, D),
        ln_gamma.reshape(1, D), ln_beta.reshape(1, D))


# speedup vs baseline: 3.5812x; 3.5812x over previous
import functools

import jax
import jax.numpy as jnp
from jax import lax
from jax.experimental import pallas as pl
from jax.experimental.pallas import tpu as pltpu
from jax.experimental.pallas import tpu_sc as plsc

N = 10000
E_MESH = 320000
E_WORLD = 160000
D = 128
H = 128

NC = 2
NS = 16
L = 16
ACC_ROWS = 10240
ROWS_PER_TILE = ACC_ROWS // NS
CNT_ROWS = ACC_ROWS // D
CHUNK = 128

MESH_PER_TILE = E_MESH // NS
WORLD_PER_TILE = E_WORLD // NS


def _sc_scatter(mesh_attr, mesh_dst, world_attr, world_dst, zeros_init):
    mesh = plsc.VectorSubcoreMesh(
        core_axis_name="c", subcore_axis_name="s", num_cores=NC, num_subcores=NS
    )

    @functools.partial(
        pl.kernel,
        out_type=(
            jax.ShapeDtypeStruct((ACC_ROWS, D), jnp.float32),
            jax.ShapeDtypeStruct((CNT_ROWS, D), jnp.float32),
            jax.ShapeDtypeStruct((ACC_ROWS, D), jnp.float32),
            jax.ShapeDtypeStruct((CNT_ROWS, D), jnp.float32),
        ),
        mesh=mesh,
        compiler_params=pltpu.CompilerParams(needs_layout_passes=False),
        scratch_types=[
            pltpu.VMEM_SHARED((ACC_ROWS, D), jnp.float32),
            pltpu.VMEM_SHARED((CNT_ROWS, D), jnp.float32),
            pltpu.VMEM((CHUNK, D), jnp.float32),
            pltpu.VMEM((CHUNK,), jnp.int32),
            pltpu.VMEM((32,), jnp.int32),
            pltpu.VMEM((16,), jnp.int32),
            pltpu.VMEM((CNT_ROWS, D), jnp.float32),
            pltpu.VMEM((CNT_ROWS,), jnp.int32),
        ],
    )
    def k(mesh_attr_h, mesh_dst_h, world_attr_h, world_dst_h, zeros_h,
          mesh_out, mesh_cnt_out, world_out, world_cnt_out,
          acc_sh, cnt_sh, buf, idx, idx_r32, idx_r16, cnt_local, idx80):
        c = lax.axis_index("c")
        s = lax.axis_index("s")
        row0 = s * ROWS_PER_TILE

        pltpu.sync_copy(zeros_h, acc_sh.at[pl.ds(row0, ROWS_PER_TILE)])
        pltpu.sync_copy(zeros_h.at[pl.ds(0, CNT_ROWS)], cnt_local)

        @pl.when(s == 0)
        def _():
            pltpu.sync_copy(zeros_h.at[pl.ds(0, CNT_ROWS)], cnt_sh)

        for j in range(CNT_ROWS // L):
            idx80[pl.ds(j * L, L)] = lax.iota(jnp.int32, L) + (j * L)
        plsc.subcore_barrier()

        ones = jnp.full((L,), 1.0, jnp.float32)

        def count_chunk(idx_ref, n):
            for j in range(n // L):
                iv = idx_ref[pl.ds(j * L, L)]
                cnts, last = plsc.scan_count(iv)
                plsc.addupdate_scatter(
                    cnt_local,
                    [iv >> 7, iv & 127],
                    cnts.astype(jnp.float32),
                    mask=last,
                )

        def edge_loop(attr_h, dst_h, per_tile, idx_rem):
            base = s * per_tile
            n_full = per_tile // CHUNK
            rem = per_tile % CHUNK

            def body(i, carry):
                off = base + i * CHUNK
                pltpu.sync_copy(attr_h.at[pl.ds(off, CHUNK)], buf)
                pltpu.sync_copy(dst_h.at[pl.ds(off, CHUNK)], idx)
                pltpu.sync_copy(buf, acc_sh.at[idx], add=True)
                count_chunk(idx, CHUNK)
                return carry

            lax.fori_loop(0, n_full, body, 0)
            if rem:
                off = base + n_full * CHUNK
                pltpu.sync_copy(attr_h.at[pl.ds(off, rem)],
                                buf.at[pl.ds(0, rem)])
                pltpu.sync_copy(dst_h.at[pl.ds(off, rem)], idx_rem)
                pltpu.sync_copy(buf.at[pl.ds(0, rem)], acc_sh.at[idx_rem],
                                add=True)
                count_chunk(idx_rem, rem)

        @pl.when(c == 0)
        def _():
            edge_loop(mesh_attr_h, mesh_dst_h, MESH_PER_TILE, idx_r32)

        @pl.when(c == 1)
        def _():
            edge_loop(world_attr_h, world_dst_h, WORLD_PER_TILE, idx_r16)

        pltpu.sync_copy(cnt_local, cnt_sh.at[idx80], add=True)
        plsc.subcore_barrier()

        @pl.when(c == 0)
        def _():
            pltpu.sync_copy(acc_sh.at[pl.ds(row0, ROWS_PER_TILE)],
                            mesh_out.at[pl.ds(row0, ROWS_PER_TILE)])

            @pl.when(s == 0)
            def _():
                pltpu.sync_copy(cnt_sh, mesh_cnt_out)

        @pl.when(c == 1)
        def _():
            pltpu.sync_copy(acc_sh.at[pl.ds(row0, ROWS_PER_TILE)],
                            world_out.at[pl.ds(row0, ROWS_PER_TILE)])

            @pl.when(s == 0)
            def _():
                pltpu.sync_copy(cnt_sh, world_cnt_out)

    return k(mesh_attr, mesh_dst, world_attr, world_dst, zeros_init)


def _tc_body(x_ref, am_ref, ac_ref, aw_ref, wc_ref, w1_ref, b1_ref, w2_ref,
             b2_ref, g_ref, be_ref, o_ref):
    xv = x_ref[...]
    am = am_ref[...] * (1.0 / jnp.maximum(ac_ref[...], 1.0))
    aw = aw_ref[...] * (1.0 / jnp.maximum(wc_ref[...], 1.0))
    w1 = w1_ref[...]
    hp = jax.lax.Precision.HIGHEST
    h1 = (jnp.dot(xv, w1[0:D], precision=hp)
          + jnp.dot(am, w1[D:2 * D], precision=hp)
          + jnp.dot(aw, w1[2 * D:3 * D], precision=hp)
          + b1_ref[...])
    h1 = jnp.maximum(h1, 0.0)
    h2 = jnp.dot(h1, w2_ref[...], precision=hp) + b2_ref[...]
    mu = jnp.mean(h2, axis=1, keepdims=True)
    dlt = h2 - mu
    var = jnp.mean(dlt * dlt, axis=1, keepdims=True)
    o_ref[...] = xv + dlt * jax.lax.rsqrt(var + 1e-5) * g_ref[...] + be_ref[...]


def _tc_mlp(x, mesh_acc, mesh_cnt, world_acc, world_cnt, W1, b1, W2, b2, g, b):
    BR = 512
    grid = (ACC_ROWS // BR,)
    full = lambda shape: pl.BlockSpec(shape, lambda i: (0, 0))
    return pl.pallas_call(
        _tc_body,
        grid=grid,
        in_specs=[
            pl.BlockSpec((BR, D), lambda i: (i, 0)),
            pl.BlockSpec((BR, D), lambda i: (i, 0)),
            pl.BlockSpec((BR, 1), lambda i: (i, 0)),
            pl.BlockSpec((BR, D), lambda i: (i, 0)),
            pl.BlockSpec((BR, 1), lambda i: (i, 0)),
            full((3 * D, H)),
            full((1, H)),
            full((H, D)),
            full((1, D)),
            full((1, D)),
            full((1, D)),
        ],
        out_specs=pl.BlockSpec((BR, D), lambda i: (i, 0)),
        out_shape=jax.ShapeDtypeStruct((N, D), jnp.float32),
    )(x, mesh_acc, mesh_cnt, world_acc, world_cnt, W1, b1, W2, b2, g, b)


def kernel(x, mesh_edge_attr, mesh_edge_index, world_edge_attr,
           world_edge_index, W1, b1, W2, b2, ln_gamma, ln_beta):
    mesh_dst = mesh_edge_index[1].astype(jnp.int32)
    world_dst = world_edge_index[1].astype(jnp.int32)
    zeros_init = jnp.zeros((ROWS_PER_TILE, D), jnp.float32)

    mesh_acc, mesh_cnt, world_acc, world_cnt = _sc_scatter(
        mesh_edge_attr, mesh_dst, world_edge_attr, world_dst, zeros_init)

    return _tc_mlp(
        x,
        mesh_acc, mesh_cnt.reshape(ACC_ROWS, 1),
        world_acc, world_cnt.reshape(ACC_ROWS, 1),
        W1, b1.reshape(1, H), W2, b2.reshape(1, D),
        ln_gamma.reshape(1, D), ln_beta.reshape(1, D))
